# Initial kernel scaffold; baseline (speedup 1.0000x reference)
import jax
import jax.numpy as jnp
from jax.experimental import pallas as pl


def kernel(hidden, gate_weight, fc1_w, fc2_w, choosed_experts):
    raise NotImplementedError("write your pallas kernel here")


# speedup vs baseline: 2.3730x; 2.3730x over previous
import functools

import jax
import jax.numpy as jnp
from jax import lax
from jax.experimental import pallas as pl
from jax.experimental.pallas import tpu as pltpu

E = 8
TOPK = 2
T = 2048
D_MODEL = 1024
D_FF = 4096

ROW_TILE = 256
S = T * TOPK + E * ROW_TILE
N_TILES = S // ROW_TILE
N_FLAT = T * TOPK
RB = 512
FF_CHUNK = 1024


def _routing_body(flat_e_ref, pos_ref, te_ref):
    lane_e = lax.broadcasted_iota(jnp.int32, (RB, E), 1)
    tri = (
        lax.broadcasted_iota(jnp.int32, (RB, RB), 0)
        > lax.broadcasted_iota(jnp.int32, (RB, RB), 1)
    ).astype(jnp.float32)

    def rank_step(b, acc):
        eb = flat_e_ref[pl.ds(b * RB, RB), :]
        oh = (eb == lane_e).astype(jnp.float32)
        within = jax.lax.dot_general(
            tri, oh, (((1,), (0,)), ((), ())),
            preferred_element_type=jnp.float32)
        rank = within + acc
        pos_ref[pl.ds(b * RB, RB), :] = rank
        return acc + jnp.sum(oh, axis=0, keepdims=True)

    counts = lax.fori_loop(0, N_FLAT // RB, rank_step,
                           jnp.zeros((1, E), jnp.float32))

    padded = jnp.ceil(counts / ROW_TILE) * ROW_TILE
    triu = (
        lax.broadcasted_iota(jnp.int32, (E, E), 0)
        < lax.broadcasted_iota(jnp.int32, (E, E), 1)
    ).astype(jnp.float32)
    base = jax.lax.dot_general(
        padded, triu, (((1,), (0,)), ((), ())),
        preferred_element_type=jnp.float32)

    def pos_step(b, _):
        eb = flat_e_ref[pl.ds(b * RB, RB), :]
        oh = (eb == lane_e).astype(jnp.float32)
        rank = pos_ref[pl.ds(b * RB, RB), :]
        p = jnp.sum((rank + base) * oh, axis=1, keepdims=True)
        pos_ref[pl.ds(b * RB, RB), :] = jnp.broadcast_to(p, (RB, E))
        return 0

    lax.fori_loop(0, N_FLAT // RB, pos_step, 0)

    s_start = lax.broadcasted_iota(jnp.int32, (128, 1), 0).astype(jnp.float32) * ROW_TILE
    m = (s_start >= base).astype(jnp.int32)
    te = jnp.sum(m, axis=1, keepdims=True) - 1
    te_ref[...] = jnp.broadcast_to(te, (128, E))


def _routing(choosed_experts):
    flat_e = choosed_experts.reshape(N_FLAT, 1).astype(jnp.int32)
    pos_f, te = pl.pallas_call(
        _routing_body,
        out_shape=(
            jax.ShapeDtypeStruct((N_FLAT, E), jnp.float32),
            jax.ShapeDtypeStruct((128, E), jnp.int32),
        ),
    )(flat_e)
    pos = pos_f[:, 0].astype(jnp.int32)
    tile_expert = te[:N_TILES, 0]
    return pos, tile_expert


def _gemm_body(te_ref, x_ref, w1_ref, w2_ref, g_ref, y_ref):
    x = x_ref[...].astype(jnp.bfloat16)
    acc = jnp.zeros((ROW_TILE, D_MODEL), jnp.float32)
    for c in range(D_FF // FF_CHUNK):
        w1c = w1_ref[0, pl.ds(c * FF_CHUNK, FF_CHUNK), :]
        h = jax.lax.dot_general(
            x, w1c, (((1,), (1,)), ((), ())),
            preferred_element_type=jnp.float32)
        h = jax.nn.gelu(h).astype(jnp.bfloat16)
        w2c = w2_ref[0, :, pl.ds(c * FF_CHUNK, FF_CHUNK)]
        acc = acc + jax.lax.dot_general(
            h, w2c, (((1,), (1,)), ((), ())),
            preferred_element_type=jnp.float32)
    y_ref[...] = acc * g_ref[...]


def _grouped_gemm(x_sorted, fc1_w, fc2_w, gate_sorted, tile_expert):
    grid_spec = pltpu.PrefetchScalarGridSpec(
        num_scalar_prefetch=1,
        grid=(N_TILES,),
        in_specs=[
            pl.BlockSpec((ROW_TILE, D_MODEL), lambda i, te: (i, 0)),
            pl.BlockSpec((1, D_FF, D_MODEL), lambda i, te: (te[i], 0, 0)),
            pl.BlockSpec((1, D_MODEL, D_FF), lambda i, te: (te[i], 0, 0)),
            pl.BlockSpec((ROW_TILE, 1), lambda i, te: (i, 0)),
        ],
        out_specs=pl.BlockSpec((ROW_TILE, D_MODEL), lambda i, te: (i, 0)),
    )
    return pl.pallas_call(
        _gemm_body,
        grid_spec=grid_spec,
        out_shape=jax.ShapeDtypeStruct((S, D_MODEL), jnp.float32),
    )(tile_expert, x_sorted, fc1_w, fc2_w, gate_sorted)


def kernel(hidden, gate_weight, fc1_w, fc2_w, choosed_experts):
    pos, tile_expert = _routing(choosed_experts)

    tok_ids = jnp.arange(N_FLAT, dtype=jnp.int32) // TOPK
    src_flat = jnp.zeros((S,), jnp.int32).at[pos].set(
        jnp.arange(N_FLAT, dtype=jnp.int32))
    src_tok = src_flat // TOPK
    x_sorted = hidden[src_tok]
    gate_sorted = gate_weight.reshape(-1)[src_flat].reshape(S, 1)

    y = _grouped_gemm(x_sorted, fc1_w.astype(jnp.bfloat16),
                      fc2_w.astype(jnp.bfloat16), gate_sorted, tile_expert)

    pos2 = pos.reshape(T, TOPK)
    out = y[pos2[:, 0]] + y[pos2[:, 1]]
    return out
